# baseline (speedup 1.0000x reference)
import functools
import math

import jax
import jax.numpy as jnp
from jax import lax
from jax.experimental import pallas as pl
from jax.experimental.pallas import tpu as pltpu
from jax.experimental.pallas import tpu_sc as plsc

_B, _L, _DM, _H, _NL, _DI = 2, 2048, 768, 12, 2, 3072
_DH = _DM // _H
_U = 40
_UPAD = 128
_BH = _B * _H
_LBLK = 512
_NLB = _L // _LBLK
_RBLK = 512
_NRB = (_B * _L) // _RBLK
_SCALE = 1.0 / math.sqrt(_DH)
_NEG = -1e30


def _dot(a, b):
    return jnp.dot(a, b, preferred_element_type=jnp.float32)


def _qkv_body(x_ref, wq_ref, wk_ref, wv_ref, bq_ref, bk_ref, bv_ref,
              q_ref, kt_ref, v_ref):
    xb = x_ref[0]
    qf = _dot(xb, wq_ref[...]) + bq_ref[...]
    kf = _dot(xb, wk_ref[...]) + bk_ref[...]
    vf = _dot(xb, wv_ref[...]) + bv_ref[...]
    for h in range(_H):
        sl = slice(h * _DH, (h + 1) * _DH)
        q_ref[h] = qf[:, sl]
        kt_ref[h] = jnp.swapaxes(kf[:, sl], 0, 1)
        v_ref[h] = vf[:, sl]


def _qkv(x3, wq, wk, wv, bq, bk, bv):
    grid = (_B, _L // _RBLK)
    full = pl.BlockSpec((_DM, _DM), lambda b, i: (0, 0))
    bias = pl.BlockSpec((1, _DM), lambda b, i: (0, 0))
    out = jax.ShapeDtypeStruct((_BH, _L, _DH), jnp.float32)
    outT = jax.ShapeDtypeStruct((_BH, _DH, _L), jnp.float32)
    ospec = pl.BlockSpec((_H, _RBLK, _DH), lambda b, i: (b, i, 0))
    ospecT = pl.BlockSpec((_H, _DH, _RBLK), lambda b, i: (b, 0, i))
    return pl.pallas_call(
        _qkv_body,
        grid=grid,
        in_specs=[pl.BlockSpec((1, _RBLK, _DM), lambda b, i: (b, i, 0)),
                  full, full, full, bias, bias, bias],
        out_specs=[ospec, ospecT, ospec],
        out_shape=[out, outT, out],
    )(x3, wq, wk, wv, bq[None], bk[None], bv[None])


_NTILES = 32
_RPT = _L // _NTILES
_CHUNK = 8
_IPAD = 48


def _cnt_body(idx_hbm, cnt_hbm, idx_v, buf):
    wid = lax.axis_index("s") * 2 + lax.axis_index("c")
    lane = lax.iota(jnp.int32, 16)
    ones = jnp.ones((16,), jnp.float32)
    zeros = jnp.zeros((16,), jnp.float32)
    tail_mask = lane < (_U - 32)
    full_mask = lane < 16

    for layer in range(_NL):
        base = layer * _L + wid * _RPT
        pltpu.sync_copy(idx_hbm.at[pl.ds(base * _IPAD, _RPT * _IPAD)], idx_v)

        def chunk_body(ci, carry):
            for r in range(_CHUNK):
                roff = r * _L
                for v in range(_L // 16):
                    buf[pl.ds(roff + v * 16, 16)] = zeros
            for r in range(_CHUNK):
                row = ci * _CHUNK + r
                roff = r * _L
                for v in range(3):
                    ii = idx_v[pl.ds(row * _IPAD + v * 16, 16)] + roff
                    mk = full_mask if v < 2 else tail_mask
                    plsc.addupdate_scatter(buf, [ii], ones, mask=mk)
            pltpu.sync_copy(
                buf,
                cnt_hbm.at[pl.ds((base + ci * _CHUNK) * _L, _CHUNK * _L)])
            return carry

        lax.fori_loop(0, _RPT // _CHUNK, chunk_body, 0)


def _cnt_build(idx_flat):
    mesh = plsc.VectorSubcoreMesh(core_axis_name="c", subcore_axis_name="s")
    out = pl.kernel(
        _cnt_body,
        mesh=mesh,
        out_type=jax.ShapeDtypeStruct((_NL * _L * _L,), jnp.float32),
        scratch_types=[
            pltpu.VMEM((_RPT * _IPAD,), jnp.int32),
            pltpu.VMEM((_CHUNK * _L,), jnp.float32),
        ],
        compiler_params=pltpu.CompilerParams(needs_layout_passes=False),
    )(idx_flat)
    return out


_MBH = 2


def _m_body(q_ref, k_ref, c_ref, m_ref):
    j = pl.program_id(1)
    nsp = 2
    part = _L // nsp
    c_parts = [c_ref[:, p * part:(p + 1) * part] for p in range(nsp)]
    for g in range(_MBH):
        qb = q_ref[g]
        smaxs, ssums = [], []
        for p in range(nsp):
            sl = slice(p * part, (p + 1) * part)
            s = _dot(qb, k_ref[j * _MBH + g][:, sl])
            c = c_parts[p]
            smaxs.append(jnp.max(jnp.where(c > 0.0, s, _NEG), axis=1))
            ssums.append(jnp.sum(s * c, axis=1))
        smax = functools.reduce(jnp.maximum, smaxs)
        ssum = functools.reduce(jnp.add, ssums)
        m_ref[g, 0] = (smax - ssum * (1.0 / _L))[None, :]


def _m_stage(q_t, k_t, cnt, layer):
    grid = (_NLB, _BH // _MBH)
    loff = layer * (_L // _LBLK)
    out = pl.pallas_call(
        _m_body,
        grid=grid,
        in_specs=[
            pl.BlockSpec((_MBH, _LBLK, _DH), lambda i, j: (j, i, 0)),
            pl.BlockSpec((_BH, _DH, _L), lambda i, j: (0, 0, 0)),
            pl.BlockSpec((_LBLK, _L), lambda i, j: (loff + i, 0)),
        ],
        out_specs=pl.BlockSpec((_MBH, 1, 1, _LBLK),
                               lambda i, j: (j, i, 0, 0)),
        out_shape=jax.ShapeDtypeStruct((_BH, _NLB, 1, _LBLK), jnp.float32),
    )(q_t, k_t, cnt)
    return out.reshape(_BH, _L)


def _m_stage_call(q_t, k_t, cnts_flat, layer):
    return _m_stage(q_t, k_t, cnts_flat.reshape(_NL * _L, _L), layer)


def _topk_body(m_ref, out_ref, ms_ref):
    ms_ref[...] = m_ref[...]
    ng = 2
    hb = _BH // ng
    iota = lax.broadcasted_iota(jnp.int32, (hb, _L), 1)
    cols = [[] for _ in range(ng)]
    ms = [ms_ref[g * hb:(g + 1) * hb, :] for g in range(ng)]
    for _ in range(_U):
        for g in range(ng):
            m = ms[g]
            rmax = jnp.max(m, axis=1, keepdims=True)
            cand = jnp.where(m == rmax, iota, _L)
            amin = jnp.min(cand, axis=1, keepdims=True)
            cols[g].append(amin)
            ms[g] = jnp.where(iota == amin, _NEG, m)
    pad = jnp.zeros((hb, _UPAD - _U), jnp.int32)
    for g in range(ng):
        out_ref[g * hb:(g + 1) * hb, 0, :] = jnp.concatenate(
            cols[g] + [pad], axis=1)


def _topk(m):
    return pl.pallas_call(
        _topk_body,
        grid=(1,),
        in_specs=[pl.BlockSpec((_BH, _L), lambda i: (0, 0))],
        out_specs=pl.BlockSpec((_BH, 1, _UPAD), lambda i: (0, 0, 0)),
        out_shape=jax.ShapeDtypeStruct((_BH, 1, _UPAD), jnp.int32),
        scratch_shapes=[pltpu.VMEM((_BH, _L), jnp.float32)],
    )(m)


_ABH = 4


def _attn_body(q_ref, k_ref, v_ref, idx_ref, ctx_ref):
    iota = lax.broadcasted_iota(jnp.int32, (_L, _U), 0)
    for g in range(_ABH):
        q = q_ref[g]
        kt = k_ref[g]
        v = v_ref[g]
        sel = idx_ref[g][:, :_U]
        oh = (iota == sel).astype(jnp.float32)
        qr = lax.dot_general(oh, q, (((0,), (0,)), ((), ())),
                             preferred_element_type=jnp.float32)
        scores = _dot(qr, kt) * _SCALE
        rmax = jnp.max(scores, axis=1, keepdims=True)
        p = jnp.exp(scores - rmax)
        attn = p / jnp.sum(p, axis=1, keepdims=True)
        upd = _dot(attn, v)
        meanv = jnp.sum(v, axis=0, keepdims=True) * (1.0 / _L)
        scat = _dot(oh, upd)
        picked = jnp.sum(oh, axis=1, keepdims=True) > 0.0
        ctx_ref[g] = jnp.where(picked, scat,
                               jnp.broadcast_to(meanv, (_L, _DH)))


def _attn(q_t, kT_t, v_t, idx_top):
    spec = pl.BlockSpec((_ABH, _L, _DH), lambda j: (j, 0, 0))
    specT = pl.BlockSpec((_ABH, _DH, _L), lambda j: (j, 0, 0))
    return pl.pallas_call(
        _attn_body,
        grid=(_BH // _ABH,),
        in_specs=[spec, specT, spec,
                  pl.BlockSpec((_ABH, 1, _UPAD), lambda j: (j, 0, 0))],
        out_specs=spec,
        out_shape=jax.ShapeDtypeStruct((_BH, _L, _DH), jnp.float32),
    )(q_t, kT_t, v_t, idx_top)


def _ln(h, g, b):
    mean = jnp.mean(h, axis=1, keepdims=True)
    var = jnp.mean((h - mean) ** 2, axis=1, keepdims=True)
    return (h - mean) / jnp.sqrt(var + 1e-5) * g + b


def _mlp_body(ctx_ref, x_ref, wo_ref, bo_ref, g1_ref, b1g_ref,
              w1_ref, b1_ref, w2_ref, b2_ref, g2_ref, b2g_ref,
              gf_ref, bf_ref, out_ref, *, final):
    cat = jnp.concatenate([ctx_ref[h] for h in range(_H)], axis=1)
    xb = x_ref[0] + _dot(cat, wo_ref[...]) + bo_ref[...]
    xb = _ln(xb, g1_ref[...], b1g_ref[...])
    y = jnp.maximum(_dot(xb, w1_ref[...]) + b1_ref[...], 0.0)
    z = _dot(y, w2_ref[...]) + b2_ref[...]
    h = _ln(xb + z, g2_ref[...], b2g_ref[...])
    if final:
        h = _ln(h, gf_ref[...], bf_ref[...])
    out_ref[0] = h


def _mlp_block(ctx_t, x3, wo, bo, g1, b1g, w1, b1, w2, b2, g2, b2g,
               gf, bf, final):
    vd = pl.BlockSpec((1, _DM), lambda bb, i: (0, 0))
    vi = pl.BlockSpec((1, _DI), lambda bb, i: (0, 0))
    return pl.pallas_call(
        functools.partial(_mlp_body, final=final),
        grid=(_B, _L // _RBLK),
        in_specs=[
            pl.BlockSpec((_H, _RBLK, _DH), lambda bb, i: (bb, i, 0)),
            pl.BlockSpec((1, _RBLK, _DM), lambda bb, i: (bb, i, 0)),
            pl.BlockSpec((_DM, _DM), lambda bb, i: (0, 0)), vd,
            vd, vd,
            pl.BlockSpec((_DM, _DI), lambda bb, i: (0, 0)), vi,
            pl.BlockSpec((_DI, _DM), lambda bb, i: (0, 0)), vd,
            vd, vd, vd, vd,
        ],
        out_specs=pl.BlockSpec((1, _RBLK, _DM), lambda bb, i: (bb, i, 0)),
        out_shape=jax.ShapeDtypeStruct((_B, _L, _DM), jnp.float32),
    )(ctx_t, x3, wo, bo[None], g1[None], b1g[None], w1, b1[None],
      w2, b2[None], g2[None], b2g[None], gf[None], bf[None])


def kernel(x, Wq, bq, Wk, bk, Wv, bv, Wo, bo, W1, b1, W2, b2,
           ln1_g, ln1_b, ln2_g, ln2_b, lnf_g, lnf_b):
    x3 = x
    idx_all = []
    for l in range(_NL):
        skey = jax.random.fold_in(jax.random.key(42), l)
        idx = jax.random.randint(skey, (_L, _U), 0, _L)
        idx_all.append(jnp.pad(idx, ((0, 0), (0, _IPAD - _U))))
    cnts = _cnt_build(jnp.stack(idx_all).reshape(-1))
    for l in range(_NL):
        q_t, kT_t, v_t = _qkv(x3, Wq[l], Wk[l], Wv[l], bq[l], bk[l], bv[l])
        m = _m_stage_call(q_t, kT_t, cnts, l)
        idx_top = _topk(m)
        ctx_t = _attn(q_t, kT_t, v_t, idx_top)
        x3 = _mlp_block(ctx_t, x3, Wo[l], bo[l], ln1_g[l], ln1_b[l],
                        W1[l], b1[l], W2[l], b2[l], ln2_g[l], ln2_b[l],
                        lnf_g, lnf_b, final=(l == _NL - 1))
    return x3


# speedup vs baseline: 1.0092x; 1.0092x over previous
import functools
import math

import jax
import jax.numpy as jnp
from jax import lax
from jax.experimental import pallas as pl
from jax.experimental.pallas import tpu as pltpu
from jax.experimental.pallas import tpu_sc as plsc

_B, _L, _DM, _H, _NL, _DI = 2, 2048, 768, 12, 2, 3072
_DH = _DM // _H
_U = 40
_UPAD = 128
_BH = _B * _H
_LBLK = 512
_NLB = _L // _LBLK
_RBLK = 512
_NRB = (_B * _L) // _RBLK
_SCALE = 1.0 / math.sqrt(_DH)
_NEG = -1e30


def _dot(a, b):
    return jnp.dot(a, b, preferred_element_type=jnp.float32)


def _qkv_body(x_ref, wq_ref, wk_ref, wv_ref, bq_ref, bk_ref, bv_ref,
              q_ref, kt_ref, v_ref):
    xb = x_ref[0]
    qf = _dot(xb, wq_ref[...]) + bq_ref[...]
    kf = _dot(xb, wk_ref[...]) + bk_ref[...]
    vf = _dot(xb, wv_ref[...]) + bv_ref[...]
    for h in range(_H):
        sl = slice(h * _DH, (h + 1) * _DH)
        q_ref[h] = qf[:, sl]
        kt_ref[h] = jnp.swapaxes(kf[:, sl], 0, 1)
        v_ref[h] = vf[:, sl]


def _qkv(x3, wq, wk, wv, bq, bk, bv):
    grid = (_B, _L // _RBLK)
    full = pl.BlockSpec((_DM, _DM), lambda b, i: (0, 0))
    bias = pl.BlockSpec((1, _DM), lambda b, i: (0, 0))
    out = jax.ShapeDtypeStruct((_BH, _L, _DH), jnp.float32)
    outT = jax.ShapeDtypeStruct((_BH, _DH, _L), jnp.float32)
    ospec = pl.BlockSpec((_H, _RBLK, _DH), lambda b, i: (b, i, 0))
    ospecT = pl.BlockSpec((_H, _DH, _RBLK), lambda b, i: (b, 0, i))
    return pl.pallas_call(
        _qkv_body,
        grid=grid,
        in_specs=[pl.BlockSpec((1, _RBLK, _DM), lambda b, i: (b, i, 0)),
                  full, full, full, bias, bias, bias],
        out_specs=[ospec, ospecT, ospec],
        out_shape=[out, outT, out],
    )(x3, wq, wk, wv, bq[None], bk[None], bv[None])


_NTILES = 32
_RPT = _L // _NTILES
_CHUNK = 8
_IPAD = 48


def _cnt_body(idx_hbm, cnt_hbm, idx_v, buf):
    wid = lax.axis_index("s") * 2 + lax.axis_index("c")
    lane = lax.iota(jnp.int32, 16)
    ones = jnp.ones((16,), jnp.float32)
    zeros = jnp.zeros((16,), jnp.float32)
    tail_mask = lane < (_U - 32)
    full_mask = lane < 16

    for layer in range(_NL):
        base = layer * _L + wid * _RPT
        pltpu.sync_copy(idx_hbm.at[pl.ds(base * _IPAD, _RPT * _IPAD)], idx_v)

        def chunk_body(ci, carry):
            for r in range(_CHUNK):
                roff = r * _L
                for v in range(_L // 16):
                    buf[pl.ds(roff + v * 16, 16)] = zeros
            for r in range(_CHUNK):
                row = ci * _CHUNK + r
                roff = r * _L
                for v in range(3):
                    ii = idx_v[pl.ds(row * _IPAD + v * 16, 16)] + roff
                    mk = full_mask if v < 2 else tail_mask
                    plsc.addupdate_scatter(buf, [ii], ones, mask=mk)
            pltpu.sync_copy(
                buf,
                cnt_hbm.at[pl.ds((base + ci * _CHUNK) * _L, _CHUNK * _L)])
            return carry

        lax.fori_loop(0, _RPT // _CHUNK, chunk_body, 0)


def _cnt_build(idx_flat):
    mesh = plsc.VectorSubcoreMesh(core_axis_name="c", subcore_axis_name="s")
    out = pl.kernel(
        _cnt_body,
        mesh=mesh,
        out_type=jax.ShapeDtypeStruct((_NL * _L * _L,), jnp.float32),
        scratch_types=[
            pltpu.VMEM((_RPT * _IPAD,), jnp.int32),
            pltpu.VMEM((_CHUNK * _L,), jnp.float32),
        ],
        compiler_params=pltpu.CompilerParams(needs_layout_passes=False),
    )(idx_flat)
    return out


_MBH = 1


def _m_body(q_ref, k_ref, c_ref, m_ref):
    j = pl.program_id(1)
    nsp = 4
    part = _L // nsp
    for g in range(_MBH):
        qb = q_ref[g]
        smaxs, ssums = [], []
        for p in range(nsp):
            sl = slice(p * part, (p + 1) * part)
            s = _dot(qb, k_ref[j * _MBH + g][:, sl])
            c = c_ref[:, sl]
            smaxs.append(jnp.max(jnp.where(c > 0.0, s, _NEG), axis=1))
            ssums.append(jnp.sum(s * c, axis=1))
        smax = functools.reduce(jnp.maximum, smaxs)
        ssum = functools.reduce(jnp.add, ssums)
        m_ref[g, 0] = (smax - ssum * (1.0 / _L))[None, :]


def _m_stage(q_t, k_t, cnt, layer):
    grid = (_NLB, _BH // _MBH)
    loff = layer * (_L // _LBLK)
    out = pl.pallas_call(
        _m_body,
        grid=grid,
        in_specs=[
            pl.BlockSpec((_MBH, _LBLK, _DH), lambda i, j: (j, i, 0)),
            pl.BlockSpec((_BH, _DH, _L), lambda i, j: (0, 0, 0)),
            pl.BlockSpec((_LBLK, _L), lambda i, j: (loff + i, 0)),
        ],
        out_specs=pl.BlockSpec((_MBH, 1, 1, _LBLK),
                               lambda i, j: (j, i, 0, 0)),
        out_shape=jax.ShapeDtypeStruct((_BH, _NLB, 1, _LBLK), jnp.float32),
    )(q_t, k_t, cnt)
    return out.reshape(_BH, _L)


def _m_stage_call(q_t, k_t, cnts_flat, layer):
    return _m_stage(q_t, k_t, cnts_flat.reshape(_NL * _L, _L), layer)


def _topk_body(m_ref, out_ref, ms_ref):
    ms_ref[...] = m_ref[...]
    ng = 2
    hb = _BH // ng
    iota = lax.broadcasted_iota(jnp.int32, (hb, _L), 1)
    cols = [[] for _ in range(ng)]
    ms = [ms_ref[g * hb:(g + 1) * hb, :] for g in range(ng)]
    for _ in range(_U):
        for g in range(ng):
            m = ms[g]
            rmax = jnp.max(m, axis=1, keepdims=True)
            cand = jnp.where(m == rmax, iota, _L)
            amin = jnp.min(cand, axis=1, keepdims=True)
            cols[g].append(amin)
            ms[g] = jnp.where(iota == amin, _NEG, m)
    pad = jnp.zeros((hb, _UPAD - _U), jnp.int32)
    for g in range(ng):
        out_ref[g * hb:(g + 1) * hb, 0, :] = jnp.concatenate(
            cols[g] + [pad], axis=1)


def _topk(m):
    return pl.pallas_call(
        _topk_body,
        grid=(1,),
        in_specs=[pl.BlockSpec((_BH, _L), lambda i: (0, 0))],
        out_specs=pl.BlockSpec((_BH, 1, _UPAD), lambda i: (0, 0, 0)),
        out_shape=jax.ShapeDtypeStruct((_BH, 1, _UPAD), jnp.int32),
        scratch_shapes=[pltpu.VMEM((_BH, _L), jnp.float32)],
    )(m)


_ABH = 4


def _attn_body(q_ref, k_ref, v_ref, idx_ref, ctx_ref):
    iota = lax.broadcasted_iota(jnp.int32, (_L, _U), 0)
    for g in range(_ABH):
        q = q_ref[g]
        kt = k_ref[g]
        v = v_ref[g]
        sel = idx_ref[g][:, :_U]
        oh = (iota == sel).astype(jnp.float32)
        qr = lax.dot_general(oh, q, (((0,), (0,)), ((), ())),
                             preferred_element_type=jnp.float32)
        scores = _dot(qr, kt) * _SCALE
        rmax = jnp.max(scores, axis=1, keepdims=True)
        p = jnp.exp(scores - rmax)
        attn = p / jnp.sum(p, axis=1, keepdims=True)
        upd = _dot(attn, v)
        meanv = jnp.sum(v, axis=0, keepdims=True) * (1.0 / _L)
        scat = _dot(oh, upd)
        picked = jnp.sum(oh, axis=1, keepdims=True) > 0.0
        ctx_ref[g] = jnp.where(picked, scat,
                               jnp.broadcast_to(meanv, (_L, _DH)))


def _attn(q_t, kT_t, v_t, idx_top):
    spec = pl.BlockSpec((_ABH, _L, _DH), lambda j: (j, 0, 0))
    specT = pl.BlockSpec((_ABH, _DH, _L), lambda j: (j, 0, 0))
    return pl.pallas_call(
        _attn_body,
        grid=(_BH // _ABH,),
        in_specs=[spec, specT, spec,
                  pl.BlockSpec((_ABH, 1, _UPAD), lambda j: (j, 0, 0))],
        out_specs=spec,
        out_shape=jax.ShapeDtypeStruct((_BH, _L, _DH), jnp.float32),
    )(q_t, kT_t, v_t, idx_top)


def _ln(h, g, b):
    mean = jnp.mean(h, axis=1, keepdims=True)
    var = jnp.mean((h - mean) ** 2, axis=1, keepdims=True)
    return (h - mean) / jnp.sqrt(var + 1e-5) * g + b


def _mlp_body(ctx_ref, x_ref, wo_ref, bo_ref, g1_ref, b1g_ref,
              w1_ref, b1_ref, w2_ref, b2_ref, g2_ref, b2g_ref,
              gf_ref, bf_ref, out_ref, *, final):
    cat = jnp.concatenate([ctx_ref[h] for h in range(_H)], axis=1)
    xb = x_ref[0] + _dot(cat, wo_ref[...]) + bo_ref[...]
    xb = _ln(xb, g1_ref[...], b1g_ref[...])
    y = jnp.maximum(_dot(xb, w1_ref[...]) + b1_ref[...], 0.0)
    z = _dot(y, w2_ref[...]) + b2_ref[...]
    h = _ln(xb + z, g2_ref[...], b2g_ref[...])
    if final:
        h = _ln(h, gf_ref[...], bf_ref[...])
    out_ref[0] = h


def _mlp_block(ctx_t, x3, wo, bo, g1, b1g, w1, b1, w2, b2, g2, b2g,
               gf, bf, final):
    vd = pl.BlockSpec((1, _DM), lambda bb, i: (0, 0))
    vi = pl.BlockSpec((1, _DI), lambda bb, i: (0, 0))
    return pl.pallas_call(
        functools.partial(_mlp_body, final=final),
        grid=(_B, _L // _RBLK),
        in_specs=[
            pl.BlockSpec((_H, _RBLK, _DH), lambda bb, i: (bb, i, 0)),
            pl.BlockSpec((1, _RBLK, _DM), lambda bb, i: (bb, i, 0)),
            pl.BlockSpec((_DM, _DM), lambda bb, i: (0, 0)), vd,
            vd, vd,
            pl.BlockSpec((_DM, _DI), lambda bb, i: (0, 0)), vi,
            pl.BlockSpec((_DI, _DM), lambda bb, i: (0, 0)), vd,
            vd, vd, vd, vd,
        ],
        out_specs=pl.BlockSpec((1, _RBLK, _DM), lambda bb, i: (bb, i, 0)),
        out_shape=jax.ShapeDtypeStruct((_B, _L, _DM), jnp.float32),
    )(ctx_t, x3, wo, bo[None], g1[None], b1g[None], w1, b1[None],
      w2, b2[None], g2[None], b2g[None], gf[None], bf[None])


def kernel(x, Wq, bq, Wk, bk, Wv, bv, Wo, bo, W1, b1, W2, b2,
           ln1_g, ln1_b, ln2_g, ln2_b, lnf_g, lnf_b):
    x3 = x
    idx_all = []
    for l in range(_NL):
        skey = jax.random.fold_in(jax.random.key(42), l)
        idx = jax.random.randint(skey, (_L, _U), 0, _L)
        idx_all.append(jnp.pad(idx, ((0, 0), (0, _IPAD - _U))))
    cnts = _cnt_build(jnp.stack(idx_all).reshape(-1))
    for l in range(_NL):
        q_t, kT_t, v_t = _qkv(x3, Wq[l], Wk[l], Wv[l], bq[l], bk[l], bv[l])
        m = _m_stage_call(q_t, kT_t, cnts, l)
        idx_top = _topk(m)
        ctx_t = _attn(q_t, kT_t, v_t, idx_top)
        x3 = _mlp_block(ctx_t, x3, Wo[l], bo[l], ln1_g[l], ln1_b[l],
                        W1[l], b1[l], W2[l], b2[l], ln2_g[l], ln2_b[l],
                        lnf_g, lnf_b, final=(l == _NL - 1))
    return x3
